# Initial kernel scaffold; baseline (speedup 1.0000x reference)
import jax
import jax.numpy as jnp
from jax.experimental import pallas as pl


def kernel(snr_logit, ii, jj, y):
    raise NotImplementedError("write your pallas kernel here")


# speedup vs baseline: 3.6586x; 3.6586x over previous
import functools

import jax
import jax.numpy as jnp
from jax import lax
from jax.experimental import pallas as pl
from jax.experimental.pallas import tpu as pltpu
from jax.experimental.pallas import tpu_sc as plsc

NI, NJ, NK = 5000, 200, 10
NJP = 208

B_OFF = 5008
S_TOT = 55040

NC, NS = 2, 16
NW = NC * NS
CHUNK = 4096
NP = NW * CHUNK
CROWS = CHUNK // 128
NROWS = NW * CROWS
SLICE = S_TOT // NS


def _tables_body(x_ref, o_ref, dmo_ref):
    x = x_ref[...]
    lin = (lax.broadcasted_iota(jnp.int32, (8, 26), 0) * 26
           + lax.broadcasted_iota(jnp.int32, (8, 26), 1))
    valid = lin < NJ
    sn = jax.nn.sigmoid(-x)
    s = 1.0 - sn
    o = jnp.log(sn * (1.0 / NK))
    d = jnp.log(s + sn * (1.0 / NK))
    o_ref[...] = jnp.where(valid, o, 0.0)
    dmo_ref[...] = jnp.where(valid, d - o, 0.0)


_tables = pl.pallas_call(
    _tables_body,
    out_shape=[jax.ShapeDtypeStruct((8, 26), jnp.float32)] * 2,
)


def _sc_body(o_hbm, dmo_hbm, ii_hbm, jj_hbm, y_hbm, out_hbm,
             ii_v, jj_v, y_v, ov_v, dmov_v, z_v, sem, acc_sh):
    cid = lax.axis_index("c")
    sid = lax.axis_index("s")
    wid = sid * NC + cid
    rbase = wid * CROWS

    pltpu.sync_copy(ii_hbm.at[pl.ds(rbase, CROWS)], ii_v)
    pltpu.sync_copy(jj_hbm.at[pl.ds(rbase, CROWS)], jj_v)
    pltpu.sync_copy(y_hbm.at[pl.ds(rbase, CROWS)], y_v)

    def gbody(j, c):
        pltpu.async_copy(o_hbm.at[jj_v.at[j]], ov_v.at[j], sem).wait()
        pltpu.async_copy(dmo_hbm.at[jj_v.at[j]], dmov_v.at[j], sem).wait()
        return c

    lax.fori_loop(0, CROWS, gbody, 0)

    zero16 = jnp.zeros((16,), jnp.float32)

    def zbody(i, c):
        z_v[pl.ds(i * 16, 16)] = zero16
        return c

    lax.fori_loop(0, SLICE // 16, zbody, 0)
    pltpu.sync_copy(z_v, acc_sh.at[pl.ds(sid * SLICE, SLICE)])

    def cbody(j, c):
        for u in range(8):
            sl = pl.ds(u * 16, 16)
            y_v[j, sl] = B_OFF + ii_v[j, sl] * NK + y_v[j, sl]
        return c

    lax.fori_loop(0, CROWS, cbody, 0)

    plsc.subcore_barrier()

    def sbody(j, c):
        pltpu.sync_copy(ov_v.at[j], acc_sh.at[ii_v.at[j]], add=True)
        pltpu.sync_copy(dmov_v.at[j], acc_sh.at[y_v.at[j]], add=True)
        return c

    lax.fori_loop(0, CROWS, sbody, 0)

    plsc.subcore_barrier()

    pltpu.sync_copy(acc_sh.at[pl.ds(sid * SLICE, SLICE)], z_v)
    pltpu.sync_copy(z_v, out_hbm.at[pl.ds(cid * S_TOT + sid * SLICE, SLICE)])


_sc_scatter = functools.partial(
    pl.kernel,
    out_type=jax.ShapeDtypeStruct((NC * S_TOT,), jnp.float32),
    mesh=plsc.VectorSubcoreMesh(core_axis_name="c", subcore_axis_name="s"),
    scratch_types=[
        pltpu.VMEM((CROWS, 128), jnp.int32),
        pltpu.VMEM((CROWS, 128), jnp.int32),
        pltpu.VMEM((CROWS, 128), jnp.int32),
        pltpu.VMEM((CROWS, 128), jnp.float32),
        pltpu.VMEM((CROWS, 128), jnp.float32),
        pltpu.VMEM((SLICE,), jnp.float32),
        pltpu.SemaphoreType.DMA,
        pltpu.VMEM_SHARED((S_TOT,), jnp.float32),
    ],
)(_sc_body)


def _final_body(a_ref, b_ref, qz_ref, vq_ref):
    b = b_ref[0] + b_ref[1]
    a = a_ref[0] + a_ref[1]
    m = jnp.max(b, axis=-1, keepdims=True)
    e = jnp.exp(b - m)
    se = jnp.sum(e, axis=-1, keepdims=True)
    qz_ref[...] = e / se
    vq_ref[...] = a + m + jnp.log(se)


_final = pl.pallas_call(
    _final_body,
    out_shape=[jax.ShapeDtypeStruct((NI, NK), jnp.float32),
               jax.ShapeDtypeStruct((NI, 1), jnp.float32)],
)


def kernel(snr_logit, ii, jj, y):
    x = jnp.pad(snr_logit, (0, NJP - NJ)).reshape(8, 26)
    o_tab, dmo_tab = _tables(x)
    o_tab = o_tab.reshape(NJP)
    dmo_tab = dmo_tab.reshape(NJP)

    pad = NP - ii.shape[0]
    ii_p = jnp.pad(ii, (0, pad)).reshape(NROWS, 128)
    jj_p = jnp.pad(jj, (0, pad), constant_values=NJ).reshape(NROWS, 128)
    y_p = jnp.pad(y, (0, pad)).reshape(NROWS, 128)

    partials = _sc_scatter(o_tab, dmo_tab, ii_p, jj_p, y_p).reshape(NC, S_TOT)

    a2 = partials[:, :NI].reshape(NC, NI, 1)
    b2 = partials[:, B_OFF:B_OFF + NI * NK].reshape(NC, NI, NK)
    qz, vq = _final(a2, b2)
    return qz, vq.reshape(NI)


# speedup vs baseline: 3.7763x; 1.0322x over previous
import functools

import jax
import jax.numpy as jnp
from jax import lax
from jax.experimental import pallas as pl
from jax.experimental.pallas import tpu as pltpu
from jax.experimental.pallas import tpu_sc as plsc

NI, NJ, NK = 5000, 200, 10
NJP = 208

B_OFF = 5008
S_TOT = 55040

NC, NS = 2, 16
NW = NC * NS
CHUNK = 4096
NP = NW * CHUNK
CROWS = CHUNK // 128
NROWS = NW * CROWS
SLICE = S_TOT // NS


def _tables_body(x_ref, o_ref, dmo_ref):
    x = x_ref[...]
    lin = (lax.broadcasted_iota(jnp.int32, (8, 26), 0) * 26
           + lax.broadcasted_iota(jnp.int32, (8, 26), 1))
    valid = lin < NJ
    sn = jax.nn.sigmoid(-x)
    s = 1.0 - sn
    o = jnp.log(sn * (1.0 / NK))
    d = jnp.log(s + sn * (1.0 / NK))
    o_ref[...] = jnp.where(valid, o, 0.0)
    dmo_ref[...] = jnp.where(valid, d - o, 0.0)


_tables = pl.pallas_call(
    _tables_body,
    out_shape=[jax.ShapeDtypeStruct((8, 26), jnp.float32)] * 2,
)


def _sc_body(o_hbm, dmo_hbm, zrow_hbm, ii_hbm, jj_hbm, y_hbm,
             out_hbm, ii_v, jj_v, y_v, ov_v, dmov_v, z_v,
             sem_g, sem_i, sem_z, sem_s, acc_sh):
    cid = lax.axis_index("c")
    sid = lax.axis_index("s")
    wid = sid * NC + cid
    rbase = wid * CROWS

    pltpu.sync_copy(jj_hbm.at[pl.ds(rbase, CROWS)], jj_v)

    def gfire(j, c):
        pltpu.async_copy(o_hbm.at[jj_v.at[j]], ov_v.at[j], sem_g)
        pltpu.async_copy(dmo_hbm.at[jj_v.at[j]], dmov_v.at[j], sem_g)
        return c

    lax.fori_loop(0, CROWS, gfire, 0)

    cp_ii = pltpu.async_copy(ii_hbm.at[pl.ds(rbase, CROWS)], ii_v, sem_i)
    cp_y = pltpu.async_copy(y_hbm.at[pl.ds(rbase, CROWS)], y_v, sem_i)

    zero16 = jnp.zeros((16,), jnp.float32)

    def zbody(i, c):
        z_v[pl.ds(i * 16, 16)] = zero16
        return c

    lax.fori_loop(0, SLICE // 16, zbody, 0)
    cp_z = pltpu.async_copy(z_v, acc_sh.at[pl.ds(sid * SLICE, SLICE)], sem_z)

    cp_ii.wait()
    cp_y.wait()

    def cbody(j, c):
        for u in range(8):
            sl = pl.ds(u * 16, 16)
            y_v[j, sl] = B_OFF + ii_v[j, sl] * NK + y_v[j, sl]
        return c

    lax.fori_loop(0, CROWS, cbody, 0)

    cp_z.wait()
    plsc.subcore_barrier()

    pltpu.make_async_copy(zrow_hbm, ov_v, sem_g).wait()
    pltpu.make_async_copy(zrow_hbm, dmov_v, sem_g).wait()

    def sfire(j, c):
        pltpu.async_copy(ov_v.at[j], acc_sh.at[ii_v.at[j]], sem_s, add=True)
        pltpu.async_copy(dmov_v.at[j], acc_sh.at[y_v.at[j]], sem_s, add=True)
        return c

    lax.fori_loop(0, CROWS, sfire, 0)

    pltpu.make_async_copy(zrow_hbm, ov_v, sem_s).wait()
    pltpu.make_async_copy(zrow_hbm, dmov_v, sem_s).wait()

    plsc.subcore_barrier()

    pltpu.sync_copy(acc_sh.at[pl.ds(sid * SLICE, SLICE)], z_v)
    pltpu.sync_copy(z_v, out_hbm.at[pl.ds(cid * S_TOT + sid * SLICE, SLICE)])


_sc_scatter = functools.partial(
    pl.kernel,
    out_type=jax.ShapeDtypeStruct((NC * S_TOT,), jnp.float32),
    mesh=plsc.VectorSubcoreMesh(core_axis_name="c", subcore_axis_name="s"),
    scratch_types=[
        pltpu.VMEM((CROWS, 128), jnp.int32),
        pltpu.VMEM((CROWS, 128), jnp.int32),
        pltpu.VMEM((CROWS, 128), jnp.int32),
        pltpu.VMEM((CROWS, 128), jnp.float32),
        pltpu.VMEM((CROWS, 128), jnp.float32),
        pltpu.VMEM((SLICE,), jnp.float32),
        pltpu.SemaphoreType.DMA,
        pltpu.SemaphoreType.DMA,
        pltpu.SemaphoreType.DMA,
        pltpu.SemaphoreType.DMA,
        pltpu.VMEM_SHARED((S_TOT,), jnp.float32),
    ],
)(_sc_body)


def _final_body(a_ref, b_ref, qz_ref, vq_ref):
    b = b_ref[0] + b_ref[1]
    a = a_ref[0] + a_ref[1]
    m = jnp.max(b, axis=-1, keepdims=True)
    e = jnp.exp(b - m)
    se = jnp.sum(e, axis=-1, keepdims=True)
    qz_ref[...] = e / se
    vq_ref[...] = a + m + jnp.log(se)


_final = pl.pallas_call(
    _final_body,
    out_shape=[jax.ShapeDtypeStruct((NI, NK), jnp.float32),
               jax.ShapeDtypeStruct((NI, 1), jnp.float32)],
)


def kernel(snr_logit, ii, jj, y):
    x = jnp.pad(snr_logit, (0, NJP - NJ)).reshape(8, 26)
    o_tab, dmo_tab = _tables(x)
    o_tab = o_tab.reshape(NJP)
    dmo_tab = dmo_tab.reshape(NJP)

    pad = NP - ii.shape[0]
    ii_p = jnp.pad(ii, (0, pad)).reshape(NROWS, 128)
    jj_p = jnp.pad(jj, (0, pad), constant_values=NJ).reshape(NROWS, 128)
    y_p = jnp.pad(y, (0, pad)).reshape(NROWS, 128)

    zrow = jnp.zeros((CROWS, 128), jnp.float32)
    partials = _sc_scatter(o_tab, dmo_tab, zrow,
                           ii_p, jj_p, y_p).reshape(NC, S_TOT)

    a2 = partials[:, :NI].reshape(NC, NI, 1)
    b2 = partials[:, B_OFF:B_OFF + NI * NK].reshape(NC, NI, NK)
    qz, vq = _final(a2, b2)
    return qz, vq.reshape(NI)


# speedup vs baseline: 21.2573x; 5.6291x over previous
import functools

import jax
import jax.numpy as jnp
from jax import lax
from jax.experimental import pallas as pl
from jax.experimental.pallas import tpu as pltpu
from jax.experimental.pallas import tpu_sc as plsc

NI, NJ, NK = 5000, 200, 10
NJP = 208

B_OFF = 5008
S_TOT = 55040

NC, NS = 2, 16
NW = NC * NS
CHUNK = 4096
NP = NW * CHUNK
CROWS = CHUNK // 128
NROWS = NW * CROWS


def _tables_body(x_ref, o_ref, dmo_ref):
    x = x_ref[...]
    lin = (lax.broadcasted_iota(jnp.int32, (8, 26), 0) * 26
           + lax.broadcasted_iota(jnp.int32, (8, 26), 1))
    valid = lin < NJ
    sn = jax.nn.sigmoid(-x)
    s = 1.0 - sn
    o = jnp.log(sn * (1.0 / NK))
    d = jnp.log(s + sn * (1.0 / NK))
    o_ref[...] = jnp.where(valid, o, 0.0)
    dmo_ref[...] = jnp.where(valid, d - o, 0.0)


_tables = pl.pallas_call(
    _tables_body,
    out_shape=[jax.ShapeDtypeStruct((8, 26), jnp.float32)] * 2,
)


def _sc_body(o_hbm, dmo_hbm, ii_hbm, jj_hbm, y_hbm,
             out_hbm, ii_v, jj_v, y_v, tab_o, tab_dmo, acc, sem_i, sem_t):
    cid = lax.axis_index("c")
    sid = lax.axis_index("s")
    wid = sid * NC + cid
    rbase = wid * CROWS

    cp_ii = pltpu.async_copy(ii_hbm.at[pl.ds(rbase, CROWS)], ii_v, sem_i)
    cp_jj = pltpu.async_copy(jj_hbm.at[pl.ds(rbase, CROWS)], jj_v, sem_i)
    cp_y = pltpu.async_copy(y_hbm.at[pl.ds(rbase, CROWS)], y_v, sem_i)
    cp_to = pltpu.async_copy(o_hbm, tab_o, sem_t)
    cp_td = pltpu.async_copy(dmo_hbm, tab_dmo, sem_t)

    zero16 = jnp.zeros((16,), jnp.float32)

    def zbody(i, c):
        for u in range(8):
            acc[pl.ds(i * 128 + u * 16, 16)] = zero16
        return c

    lax.fori_loop(0, S_TOT // 128, zbody, 0)

    cp_ii.wait()
    cp_jj.wait()
    cp_y.wait()
    cp_to.wait()
    cp_td.wait()

    def mbody(j, c):
        for u in range(8):
            sl = pl.ds(u * 16, 16)
            ii16 = ii_v[j, sl]
            jj16 = jj_v[j, sl]
            y16 = y_v[j, sl]
            o16 = plsc.load_gather(tab_o, [jj16])
            d16 = plsc.load_gather(tab_dmo, [jj16])
            plsc.addupdate_scatter(acc, [ii16], o16)
            plsc.addupdate_scatter(acc, [B_OFF + ii16 * NK + y16], d16)
        return c

    lax.fori_loop(0, CROWS, mbody, 0)

    pltpu.sync_copy(acc, out_hbm.at[pl.ds(wid * S_TOT, S_TOT)])


_sc_scatter = functools.partial(
    pl.kernel,
    out_type=jax.ShapeDtypeStruct((NW * S_TOT,), jnp.float32),
    mesh=plsc.VectorSubcoreMesh(core_axis_name="c", subcore_axis_name="s"),
    compiler_params=pltpu.CompilerParams(needs_layout_passes=False),
    scratch_types=[
        pltpu.VMEM((CROWS, 128), jnp.int32),
        pltpu.VMEM((CROWS, 128), jnp.int32),
        pltpu.VMEM((CROWS, 128), jnp.int32),
        pltpu.VMEM((NJP,), jnp.float32),
        pltpu.VMEM((NJP,), jnp.float32),
        pltpu.VMEM((S_TOT,), jnp.float32),
        pltpu.SemaphoreType.DMA,
        pltpu.SemaphoreType.DMA,
    ],
)(_sc_body)


def _reduce_body(p_ref, r_ref):
    r_ref[...] = jnp.sum(p_ref[...], axis=0)


_reduce = pl.pallas_call(
    _reduce_body,
    out_shape=jax.ShapeDtypeStruct((S_TOT,), jnp.float32),
)


def _final_body(a_ref, b_ref, qz_ref, vq_ref):
    b = b_ref[...]
    a = a_ref[...]
    m = jnp.max(b, axis=-1, keepdims=True)
    e = jnp.exp(b - m)
    se = jnp.sum(e, axis=-1, keepdims=True)
    qz_ref[...] = e / se
    vq_ref[...] = a + m + jnp.log(se)


_final = pl.pallas_call(
    _final_body,
    out_shape=[jax.ShapeDtypeStruct((NI, NK), jnp.float32),
               jax.ShapeDtypeStruct((NI, 1), jnp.float32)],
)


def kernel(snr_logit, ii, jj, y):
    x = jnp.pad(snr_logit, (0, NJP - NJ)).reshape(8, 26)
    o_tab, dmo_tab = _tables(x)
    o_tab = o_tab.reshape(NJP)
    dmo_tab = dmo_tab.reshape(NJP)

    pad = NP - ii.shape[0]
    ii_p = jnp.pad(ii, (0, pad)).reshape(NROWS, 128)
    jj_p = jnp.pad(jj, (0, pad), constant_values=NJ).reshape(NROWS, 128)
    y_p = jnp.pad(y, (0, pad)).reshape(NROWS, 128)

    partials = _sc_scatter(o_tab, dmo_tab, ii_p, jj_p, y_p)
    red = _reduce(partials.reshape(NW, S_TOT))

    a = red[:NI].reshape(NI, 1)
    b = red[B_OFF:B_OFF + NI * NK].reshape(NI, NK)
    qz, vq = _final(a, b)
    return qz, vq.reshape(NI)


# speedup vs baseline: 29.0604x; 1.3671x over previous
import functools

import jax
import jax.numpy as jnp
from jax import lax
from jax.experimental import pallas as pl
from jax.experimental.pallas import tpu as pltpu
from jax.experimental.pallas import tpu_sc as plsc

NI, NJ, NK = 5000, 200, 10
NJP = 208

B_OFF = 5008
S_TOT = 55040

NC, NS = 2, 16
NW = NC * NS
CHUNK = 4096
NP = NW * CHUNK


def _tables_body(x_ref, o_ref, dmo_ref):
    x = x_ref[...]
    lin = (lax.broadcasted_iota(jnp.int32, (8, 26), 0) * 26
           + lax.broadcasted_iota(jnp.int32, (8, 26), 1))
    valid = lin < NJ
    sn = jax.nn.sigmoid(-x)
    s = 1.0 - sn
    o = jnp.log(sn * (1.0 / NK))
    d = jnp.log(s + sn * (1.0 / NK))
    o_ref[...] = jnp.where(valid, o, 0.0)
    dmo_ref[...] = jnp.where(valid, d - o, 0.0)


_tables = pl.pallas_call(
    _tables_body,
    out_shape=[jax.ShapeDtypeStruct((8, 26), jnp.float32)] * 2,
)


def _sc_body(o_hbm, dmo_hbm, ii_hbm, jj_hbm, y_hbm,
             out_hbm, ii_v, jj_v, y_v, tab_o, tab_dmo, acc, sem_i, sem_t):
    cid = lax.axis_index("c")
    sid = lax.axis_index("s")
    wid = sid * NC + cid
    base = wid * CHUNK

    cp_ii = pltpu.async_copy(ii_hbm.at[pl.ds(base, CHUNK)], ii_v, sem_i)
    cp_jj = pltpu.async_copy(jj_hbm.at[pl.ds(base, CHUNK)], jj_v, sem_i)
    cp_y = pltpu.async_copy(y_hbm.at[pl.ds(base, CHUNK)], y_v, sem_i)
    cp_to = pltpu.async_copy(o_hbm, tab_o, sem_t)
    cp_td = pltpu.async_copy(dmo_hbm, tab_dmo, sem_t)

    zero16 = jnp.zeros((16,), jnp.float32)

    def zbody(i, c):
        for u in range(8):
            acc[pl.ds(i * 128 + u * 16, 16)] = zero16
        return c

    lax.fori_loop(0, S_TOT // 128, zbody, 0)

    cp_ii.wait()
    cp_jj.wait()
    cp_y.wait()
    cp_to.wait()
    cp_td.wait()

    def mbody(j, c):
        for u in range(8):
            sl = pl.ds(j * 128 + u * 16, 16)
            ii16 = ii_v[sl]
            jj16 = jj_v[sl]
            y16 = y_v[sl]
            o16 = plsc.load_gather(tab_o, [jj16])
            d16 = plsc.load_gather(tab_dmo, [jj16])
            plsc.addupdate_scatter(acc, [ii16], o16)
            plsc.addupdate_scatter(acc, [B_OFF + y16 * NI + ii16], d16)
        return c

    lax.fori_loop(0, CHUNK // 128, mbody, 0)

    pltpu.sync_copy(acc, out_hbm.at[wid])


_sc_scatter = functools.partial(
    pl.kernel,
    out_type=jax.ShapeDtypeStruct((NW, S_TOT), jnp.float32),
    mesh=plsc.VectorSubcoreMesh(core_axis_name="c", subcore_axis_name="s"),
    compiler_params=pltpu.CompilerParams(needs_layout_passes=False),
    scratch_types=[
        pltpu.VMEM((CHUNK,), jnp.int32),
        pltpu.VMEM((CHUNK,), jnp.int32),
        pltpu.VMEM((CHUNK,), jnp.int32),
        pltpu.VMEM((NJP,), jnp.float32),
        pltpu.VMEM((NJP,), jnp.float32),
        pltpu.VMEM((S_TOT,), jnp.float32),
        pltpu.SemaphoreType.DMA,
        pltpu.SemaphoreType.DMA,
    ],
)(_sc_body)


def _reduce_body(p_ref, r_ref):
    r_ref[...] = jnp.sum(p_ref[...], axis=0)


_reduce = pl.pallas_call(
    _reduce_body,
    out_shape=jax.ShapeDtypeStruct((S_TOT,), jnp.float32),
)


def _final_body(a_ref, b_ref, qzt_ref, vq_ref):
    b = b_ref[...]
    a = a_ref[...]
    m = jnp.max(b, axis=0, keepdims=True)
    e = jnp.exp(b - m)
    se = jnp.sum(e, axis=0, keepdims=True)
    qzt_ref[...] = e / se
    vq_ref[...] = a + m + jnp.log(se)


_final = pl.pallas_call(
    _final_body,
    out_shape=[jax.ShapeDtypeStruct((NK, NI), jnp.float32),
               jax.ShapeDtypeStruct((1, NI), jnp.float32)],
)


def kernel(snr_logit, ii, jj, y):
    x = jnp.pad(snr_logit, (0, NJP - NJ)).reshape(8, 26)
    o_tab, dmo_tab = _tables(x)
    o_tab = o_tab.reshape(NJP)
    dmo_tab = dmo_tab.reshape(NJP)

    pad = NP - ii.shape[0]
    ii_p = jnp.pad(ii, (0, pad))
    jj_p = jnp.pad(jj, (0, pad), constant_values=NJ)
    y_p = jnp.pad(y, (0, pad))

    partials = _sc_scatter(o_tab, dmo_tab, ii_p, jj_p, y_p)
    red = _reduce(partials)

    a = red[:NI].reshape(1, NI)
    b = red[B_OFF:B_OFF + NI * NK].reshape(NK, NI)
    qzt, vq = _final(a, b)
    return qzt.T, vq.reshape(NI)


# speedup vs baseline: 41.1129x; 1.4147x over previous
import functools

import jax
import jax.numpy as jnp
from jax import lax
from jax.experimental import pallas as pl
from jax.experimental.pallas import tpu as pltpu
from jax.experimental.pallas import tpu_sc as plsc

NI, NJ, NK = 5000, 200, 10
NJP = 208

B_OFF = 5008
S_TOT = 55040

NC, NS = 2, 16
NW = NC * NS
NA = 100000
CHUNK = 3120
REM = NA - NW * CHUNK
FROWS = CHUNK // 128
TAIL16 = (CHUNK - FROWS * 128) // 16
SCR = CHUNK + REM


def _tables_body(x_ref, o_ref, dmo_ref):
    x = x_ref[...]
    valid = lax.broadcasted_iota(jnp.int32, (1, NJP), 1) < NJ
    sn = jax.nn.sigmoid(-x)
    s = 1.0 - sn
    o = jnp.log(sn * (1.0 / NK))
    d = jnp.log(s + sn * (1.0 / NK))
    o_ref[...] = jnp.where(valid, o, 0.0)
    dmo_ref[...] = jnp.where(valid, d - o, 0.0)


_tables = pl.pallas_call(
    _tables_body,
    out_shape=[jax.ShapeDtypeStruct((1, NJP), jnp.float32)] * 2,
)


def _sc_body(o_hbm, dmo_hbm, ii_hbm, jj_hbm, y_hbm,
             out_hbm, ii_v, jj_v, y_v, tab_o, tab_dmo, acc, sem_i, sem_t):
    cid = lax.axis_index("c")
    sid = lax.axis_index("s")
    wid = sid * NC + cid
    base = wid * CHUNK

    cp_ii = pltpu.async_copy(ii_hbm.at[pl.ds(base, CHUNK)],
                             ii_v.at[pl.ds(0, CHUNK)], sem_i)
    cp_jj = pltpu.async_copy(jj_hbm.at[pl.ds(base, CHUNK)],
                             jj_v.at[pl.ds(0, CHUNK)], sem_i)
    cp_y = pltpu.async_copy(y_hbm.at[pl.ds(base, CHUNK)],
                            y_v.at[pl.ds(0, CHUNK)], sem_i)
    cp_ii2 = pltpu.async_copy(ii_hbm.at[pl.ds(NW * CHUNK, REM)],
                              ii_v.at[pl.ds(CHUNK, REM)], sem_i)
    cp_jj2 = pltpu.async_copy(jj_hbm.at[pl.ds(NW * CHUNK, REM)],
                              jj_v.at[pl.ds(CHUNK, REM)], sem_i)
    cp_y2 = pltpu.async_copy(y_hbm.at[pl.ds(NW * CHUNK, REM)],
                             y_v.at[pl.ds(CHUNK, REM)], sem_i)
    cp_to = pltpu.async_copy(o_hbm.at[0], tab_o, sem_t)
    cp_td = pltpu.async_copy(dmo_hbm.at[0], tab_dmo, sem_t)

    zero16 = jnp.zeros((16,), jnp.float32)

    def zbody(i, c):
        for u in range(8):
            acc[pl.ds(i * 128 + u * 16, 16)] = zero16
        return c

    lax.fori_loop(0, S_TOT // 128, zbody, 0)

    cp_ii.wait()
    cp_jj.wait()
    cp_y.wait()
    cp_ii2.wait()
    cp_jj2.wait()
    cp_y2.wait()
    cp_to.wait()
    cp_td.wait()

    def step(sl):
        ii16 = ii_v[sl]
        jj16 = jj_v[sl]
        y16 = y_v[sl]
        o16 = plsc.load_gather(tab_o, [jj16])
        d16 = plsc.load_gather(tab_dmo, [jj16])
        plsc.addupdate_scatter(acc, [ii16], o16)
        plsc.addupdate_scatter(acc, [B_OFF + y16 * NI + ii16], d16)

    def mbody(j, c):
        for u in range(8):
            step(pl.ds(j * 128 + u * 16, 16))
        return c

    lax.fori_loop(0, FROWS, mbody, 0)

    ntail = TAIL16 + jnp.where(wid == NW - 1, REM // 16, 0)

    def tbody(t, c):
        step(pl.ds(FROWS * 128 + t * 16, 16))
        return c

    lax.fori_loop(0, ntail, tbody, 0)

    pltpu.sync_copy(acc, out_hbm.at[wid])


_sc_scatter = functools.partial(
    pl.kernel,
    out_type=jax.ShapeDtypeStruct((NW, S_TOT), jnp.float32),
    mesh=plsc.VectorSubcoreMesh(core_axis_name="c", subcore_axis_name="s"),
    compiler_params=pltpu.CompilerParams(needs_layout_passes=False),
    scratch_types=[
        pltpu.VMEM((SCR,), jnp.int32),
        pltpu.VMEM((SCR,), jnp.int32),
        pltpu.VMEM((SCR,), jnp.int32),
        pltpu.VMEM((NJP,), jnp.float32),
        pltpu.VMEM((NJP,), jnp.float32),
        pltpu.VMEM((S_TOT,), jnp.float32),
        pltpu.SemaphoreType.DMA,
        pltpu.SemaphoreType.DMA,
    ],
)(_sc_body)


def _reduce_body(p_ref, r_ref):
    r_ref[...] = jnp.sum(p_ref[...], axis=0)


_reduce = pl.pallas_call(
    _reduce_body,
    out_shape=jax.ShapeDtypeStruct((S_TOT,), jnp.float32),
)


def _final_body(a_ref, b_ref, qzt_ref, vq_ref):
    b = b_ref[...]
    a = a_ref[...]
    m = jnp.max(b, axis=0, keepdims=True)
    e = jnp.exp(b - m)
    se = jnp.sum(e, axis=0, keepdims=True)
    qzt_ref[...] = e / se
    vq_ref[...] = a + m + jnp.log(se)


_final = pl.pallas_call(
    _final_body,
    out_shape=[jax.ShapeDtypeStruct((NK, NI), jnp.float32),
               jax.ShapeDtypeStruct((1, NI), jnp.float32)],
)


def kernel(snr_logit, ii, jj, y):
    x = jnp.pad(snr_logit, (0, NJP - NJ)).reshape(1, NJP)
    o_tab, dmo_tab = _tables(x)

    partials = _sc_scatter(o_tab, dmo_tab, ii, jj, y)
    red = _reduce(partials)

    a = red[:NI].reshape(1, NI)
    b = red[B_OFF:B_OFF + NI * NK].reshape(NK, NI)
    qzt, vq = _final(a, b)
    return qzt.T, vq.reshape(NI)
